# baseline (speedup 1.0000x reference)
import functools

import jax
import jax.numpy as jnp
import numpy as np
from jax import lax
from jax.experimental import pallas as pl
from jax.experimental.pallas import tpu as pltpu
from jax.experimental.pallas import tpu_sc as plsc

_LANES = 16
_N_WORKERS = 32


def _project(x, wcat, bias2, nbd):
    n, d = x.shape
    rows = 2000
    assert n % rows == 0

    def body(x_ref, w_ref, b_ref, y_ref, base_ref):
        ycat = jnp.dot(x_ref[...], w_ref[...], preferred_element_type=jnp.float32)
        y_ref[...] = ycat[:, :nbd].astype(jnp.bfloat16)
        base_ref[...] = ycat[:, nbd:] + b_ref[...]

    return pl.pallas_call(
        body,
        grid=(n // rows,),
        in_specs=[
            pl.BlockSpec((rows, d), lambda i: (i, 0)),
            pl.BlockSpec(wcat.shape, lambda i: (0, 0)),
            pl.BlockSpec((1, d), lambda i: (0, 0)),
        ],
        out_specs=[
            pl.BlockSpec((rows, nbd), lambda i: (i, 0)),
            pl.BlockSpec((rows, d), lambda i: (i, 0)),
        ],
        out_shape=(
            jax.ShapeDtypeStruct((n, nbd), jnp.bfloat16),
            jax.ShapeDtypeStruct((n, d), jnp.float32),
        ),
    )(x, wcat, bias2)


def _make_sc_edge_kernel(n, d, nb, nr, e_pad):
    per_w = e_pad // _N_WORKERS
    groups = per_w // _LANES
    nbd = nb * d
    zchunk = n // 10
    assert zchunk % 8 == 0 and per_w % 8 == 0

    mesh = plsc.VectorSubcoreMesh(core_axis_name="c", subcore_axis_name="s")

    @functools.partial(
        pl.kernel,
        mesh=mesh,
        compiler_params=pltpu.CompilerParams(needs_layout_passes=False),
        out_type=(
            jax.ShapeDtypeStruct((2, n, d), jnp.float32),
            jax.ShapeDtypeStruct((2 * n,), jnp.float32),
        ),
        scratch_types=[
            pltpu.VMEM((nr * nb,), jnp.float32),
            pltpu.VMEM((per_w,), jnp.int32),
            pltpu.VMEM((per_w,), jnp.int32),
            pltpu.VMEM((per_w,), jnp.int32),
            pltpu.VMEM((_LANES, nbd // 2), jnp.float32),
            pltpu.VMEM((_LANES, nbd // 2), jnp.float32),
            pltpu.VMEM((_LANES, d), jnp.float32),
            pltpu.VMEM((_LANES, d), jnp.float32),
            pltpu.VMEM((nb * _LANES,), jnp.float32),
            pltpu.VMEM((per_w,), jnp.float32),
            pltpu.VMEM((1008,), jnp.float32),
            pltpu.VMEM_SHARED((n + 8, d), jnp.float32),
            pltpu.VMEM_SHARED((n + 8,), jnp.float32),
            pltpu.SemaphoreType.DMA,
            pltpu.SemaphoreType.DMA,
            pltpu.SemaphoreType.DMA,
            pltpu.SemaphoreType.DMA,
        ],
    )
    def sc(y_hbm, src_hbm, dst_hbm, et_hbm, att_hbm,
           sums_out, cnt_out,
           att_v, src_v, dst_v, et_v, rows_a, rows_b, msg_a, msg_b, ab_v,
           ones_v, zflat_v, sums_sh, cnt_sh, sem_a, sem_b, sem_ma, sem_mb):
        cid = lax.axis_index("c")
        sid = lax.axis_index("s")
        w = cid * 16 + sid
        base = pl.multiple_of(w * per_w, 8)

        pltpu.sync_copy(att_hbm, att_v)
        pltpu.sync_copy(src_hbm.at[pl.ds(base, per_w)], src_v)
        pltpu.sync_copy(dst_hbm.at[pl.ds(base, per_w)], dst_v)
        pltpu.sync_copy(et_hbm.at[pl.ds(base, per_w)], et_v)

        def fill_ones(k, c):
            ones_v[pl.ds(pl.multiple_of(k * _LANES, _LANES), _LANES)] = (
                jnp.ones((_LANES,), jnp.float32))
            return c

        lax.fori_loop(0, per_w // _LANES, fill_ones, 0)

        @pl.when(sid < 10)
        def _():
            z16 = jnp.zeros((_LANES,), jnp.float32)
            for r in range(_LANES):
                for j in range(d // _LANES):
                    msg_a[r, pl.ds(j * _LANES, _LANES)] = z16

            def zflat_body(k, c):
                zflat_v[pl.ds(k * _LANES, _LANES)] = z16
                return c

            lax.fori_loop(0, 1008 // _LANES, zflat_body, 0)
            off = pl.multiple_of(sid * zchunk, 8)

            def initrows(k, c):
                o = pl.multiple_of(off + k * _LANES, 8)
                pltpu.sync_copy(msg_a, sums_sh.at[pl.ds(o, _LANES)])
                return c

            lax.fori_loop(0, zchunk // _LANES, initrows, 0)
            pltpu.sync_copy(msg_a,
                            sums_sh.at[pl.ds(off + zchunk - _LANES, _LANES)])
            pltpu.sync_copy(zflat_v, cnt_sh.at[pl.ds(off, 1008)])

        plsc.subcore_barrier()

        rows = (rows_a, rows_b)
        sems = (sem_a, sem_b)
        msgs = (msg_a, msg_b)
        msems = (sem_ma, sem_mb)

        def issue(gg, buf, s):
            gb = pl.multiple_of(gg * _LANES, _LANES)
            pltpu.async_copy(y_hbm.at[src_v[pl.ds(gb, _LANES)]], buf, s)

        def compute(gg, buf, msg):
            gb = pl.multiple_of(gg * _LANES, _LANES)
            t16 = et_v[pl.ds(gb, _LANES)]
            for b in range(nb):
                ab_v[pl.ds(b * _LANES, _LANES)] = plsc.load_gather(
                    att_v, [t16 * nb + b])
            for c in range(_LANES):
                a = [
                    plsc.load_gather(
                        ab_v,
                        [jnp.full((_LANES,), b * _LANES + c, jnp.int32)])
                    for b in range(nb)
                ]
                for j in range(d // 32):
                    acc_e = acc_o = None
                    for b in range(nb):
                        w16 = buf[c, pl.ds(b * (d // 2) + j * _LANES,
                                           _LANES)]
                        pe, po = plsc.unpack(
                            plsc.bitcast(w16, jnp.bfloat16),
                            format=plsc.PackFormat.INTERLEAVED)
                        if b == 0:
                            acc_e = a[0] * pe
                            acc_o = a[0] * po
                        else:
                            acc_e = acc_e + a[b] * pe
                            acc_o = acc_o + a[b] * po
                    msg[c, pl.ds(j * 32, _LANES)] = acc_e
                    msg[c, pl.ds(j * 32 + _LANES, _LANES)] = acc_o

        def wait_gather(b):
            pltpu.make_async_copy(
                y_hbm.at[src_v[pl.ds(0, _LANES)]], rows[b], sems[b]).wait()

        def wait_scatter(b):
            pltpu.make_async_copy(
                msgs[b], sums_sh.at[dst_v[pl.ds(0, _LANES)]],
                msems[b]).wait()

        issue(0, rows[0], sems[0])

        def pair(p, carry):
            g = p * 2
            for b in range(2):
                gg = g + b

                @pl.when(gg + 1 < groups)
                def _():
                    issue(gg + 1, rows[1 - b], sems[1 - b])

                wait_gather(b)

                @pl.when(gg >= 2)
                def _():
                    wait_scatter(b)

                compute(gg, rows[b], msgs[b])
                gb = pl.multiple_of(gg * _LANES, _LANES)
                d16 = dst_v[pl.ds(gb, _LANES)]
                pltpu.async_copy(msgs[b], sums_sh.at[d16], msems[b],
                                 add=True)
            return carry

        lax.fori_loop(0, groups // 2, pair, 0)
        wait_scatter(0)
        wait_scatter(1)
        pltpu.sync_copy(ones_v, cnt_sh.at[dst_v], add=True)
        plsc.subcore_barrier()

        @pl.when(sid < 10)
        def _():
            off = pl.multiple_of(sid * zchunk, 8)

            def outrows(k, c):
                o = pl.multiple_of(off + k * _LANES, 8)
                pltpu.sync_copy(sums_sh.at[pl.ds(o, _LANES)], msg_a)
                pltpu.sync_copy(msg_a, sums_out.at[cid, pl.ds(o, _LANES)])
                return c

            lax.fori_loop(0, zchunk // _LANES, outrows, 0)
            o2 = pl.multiple_of(off + zchunk - _LANES, 8)
            pltpu.sync_copy(sums_sh.at[pl.ds(o2, _LANES)], msg_a)
            pltpu.sync_copy(msg_a, sums_out.at[cid, pl.ds(o2, _LANES)])
            coff = pl.multiple_of(cid * n + off, 8)
            pltpu.sync_copy(cnt_sh.at[pl.ds(off, zchunk)],
                            zflat_v.at[pl.ds(0, zchunk)])
            pltpu.sync_copy(zflat_v.at[pl.ds(0, zchunk)],
                            cnt_out.at[pl.ds(coff, zchunk)])

    return sc


def _combine(base, sums, cnt3):
    n, d = base.shape
    rows = 2000
    assert n % rows == 0

    def body(base_ref, s_ref, c_ref, o_ref):
        s = s_ref[0] + s_ref[1]
        c = c_ref[0] + c_ref[1]
        o_ref[...] = base_ref[...] + s / jnp.maximum(c, 1.0)

    return pl.pallas_call(
        body,
        grid=(n // rows,),
        in_specs=[
            pl.BlockSpec((rows, d), lambda i: (i, 0)),
            pl.BlockSpec((2, rows, d), lambda i: (0, i, 0)),
            pl.BlockSpec((2, rows, 1), lambda i: (0, i, 0)),
        ],
        out_specs=pl.BlockSpec((rows, d), lambda i: (i, 0)),
        out_shape=jax.ShapeDtypeStruct((n, d), jnp.float32),
    )(base, sums, cnt3)


def kernel(x, edge_index, edge_type, basis, att, root, bias):
    n, d = x.shape
    nb = basis.shape[0]
    e = edge_type.shape[0]
    nbd = nb * d

    src = edge_index[0].astype(jnp.int32)
    dst = edge_index[1].astype(jnp.int32)
    et = edge_type.astype(jnp.int32)

    w2 = basis.transpose(1, 0, 2).reshape(d, nbd)
    blk = np.arange(32).reshape(2, 16).T.reshape(-1)
    col_perm = (np.arange(0, nbd, 32)[:, None] + blk[None, :]).reshape(-1)
    w2 = w2[:, col_perm]
    wcat = jnp.concatenate([w2, root], axis=1)
    bias2 = bias.reshape(1, d)

    y, base = _project(x, wcat, bias2, nbd)
    yp = jax.lax.bitcast_convert_type(
        y.reshape(n, nbd // 2, 2), jnp.float32)

    e_pad = -(-e // (_N_WORKERS * _LANES * 2)) * (_N_WORKERS * _LANES * 2)
    pad = e_pad - e
    src_p = jnp.concatenate([src, jnp.zeros((pad,), jnp.int32)])
    dst_p = jnp.concatenate([dst, jnp.full((pad,), n, jnp.int32)])
    et_p = jnp.concatenate([et, jnp.zeros((pad,), jnp.int32)])
    att_flat = att.reshape(-1)

    sc_fn = _make_sc_edge_kernel(n, d, nb, att.shape[0], e_pad)
    sums, cnt = sc_fn(yp, src_p, dst_p, et_p, att_flat)

    return _combine(base, sums, cnt.reshape(2, n, 1))


# speedup vs baseline: 1.7047x; 1.7047x over previous
import functools

import jax
import jax.numpy as jnp
import numpy as np
from jax import lax
from jax.experimental import pallas as pl
from jax.experimental.pallas import tpu as pltpu
from jax.experimental.pallas import tpu_sc as plsc

_LANES = 16
_N_WORKERS = 32


def _project(x, wcat, bias2, nbd):
    n, d = x.shape
    rows = 2000
    assert n % rows == 0

    def body(x_ref, w_ref, b_ref, y_ref, base_ref):
        ycat = jnp.dot(x_ref[...], w_ref[...], preferred_element_type=jnp.float32)
        lo = jax.lax.bitcast_convert_type(
            ycat[:, :nbd // 2].astype(jnp.bfloat16), jnp.uint16)
        hi = jax.lax.bitcast_convert_type(
            ycat[:, nbd // 2:nbd].astype(jnp.bfloat16), jnp.uint16)
        packed = lo.astype(jnp.uint32) | (hi.astype(jnp.uint32) << 16)
        y_ref[...] = jax.lax.bitcast_convert_type(packed, jnp.float32)
        base_ref[...] = ycat[:, nbd:] + b_ref[...]

    return pl.pallas_call(
        body,
        grid=(n // rows,),
        in_specs=[
            pl.BlockSpec((rows, d), lambda i: (i, 0)),
            pl.BlockSpec(wcat.shape, lambda i: (0, 0)),
            pl.BlockSpec((1, d), lambda i: (0, 0)),
        ],
        out_specs=[
            pl.BlockSpec((rows, nbd // 2), lambda i: (i, 0)),
            pl.BlockSpec((rows, d), lambda i: (i, 0)),
        ],
        out_shape=(
            jax.ShapeDtypeStruct((n, nbd // 2), jnp.float32),
            jax.ShapeDtypeStruct((n, d), jnp.float32),
        ),
    )(x, wcat, bias2)


def _make_sc_edge_kernel(n, d, nb, nr, e_pad):
    per_w = e_pad // _N_WORKERS
    groups = per_w // _LANES
    nbd = nb * d
    zchunk = n // 10
    assert zchunk % 8 == 0 and per_w % 8 == 0

    mesh = plsc.VectorSubcoreMesh(core_axis_name="c", subcore_axis_name="s")

    @functools.partial(
        pl.kernel,
        mesh=mesh,
        compiler_params=pltpu.CompilerParams(needs_layout_passes=False),
        out_type=(
            jax.ShapeDtypeStruct((2, n, d), jnp.float32),
            jax.ShapeDtypeStruct((2 * n,), jnp.float32),
        ),
        scratch_types=[
            pltpu.VMEM((nr * nb + _LANES,), jnp.float32),
            pltpu.VMEM((per_w,), jnp.int32),
            pltpu.VMEM((per_w,), jnp.int32),
            pltpu.VMEM((per_w,), jnp.int32),
            pltpu.VMEM((_LANES, nbd // 2), jnp.float32),
            pltpu.VMEM((_LANES, nbd // 2), jnp.float32),
            pltpu.VMEM((_LANES, d), jnp.float32),
            pltpu.VMEM((_LANES, d), jnp.float32),
            pltpu.VMEM((per_w,), jnp.float32),
            pltpu.VMEM((1008,), jnp.float32),
            pltpu.VMEM_SHARED((n + 8, d), jnp.float32),
            pltpu.VMEM_SHARED((n + 8,), jnp.float32),
            pltpu.SemaphoreType.DMA,
            pltpu.SemaphoreType.DMA,
            pltpu.SemaphoreType.DMA,
            pltpu.SemaphoreType.DMA,
        ],
    )
    def sc(y_hbm, src_hbm, dst_hbm, et_hbm, att_hbm,
           sums_out, cnt_out,
           att_v, src_v, dst_v, et_v, rows_a, rows_b, msg_a, msg_b,
           ones_v, zflat_v, sums_sh, cnt_sh, sem_a, sem_b, sem_ma, sem_mb):
        cid = lax.axis_index("c")
        sid = lax.axis_index("s")
        w = cid * 16 + sid
        base = pl.multiple_of(w * per_w, 8)

        pltpu.sync_copy(att_hbm, att_v.at[pl.ds(0, nr * nb)])
        pltpu.sync_copy(src_hbm.at[pl.ds(base, per_w)], src_v)
        pltpu.sync_copy(dst_hbm.at[pl.ds(base, per_w)], dst_v)
        pltpu.sync_copy(et_hbm.at[pl.ds(base, per_w)], et_v)

        def fill_ones(k, c):
            ones_v[pl.ds(pl.multiple_of(k * _LANES, _LANES), _LANES)] = (
                jnp.ones((_LANES,), jnp.float32))
            return c

        lax.fori_loop(0, per_w // _LANES, fill_ones, 0)

        @pl.when(sid < 10)
        def _():
            z16 = jnp.zeros((_LANES,), jnp.float32)
            for r in range(_LANES):
                for j in range(d // _LANES):
                    msg_a[r, pl.ds(j * _LANES, _LANES)] = z16

            def zflat_body(k, c):
                zflat_v[pl.ds(k * _LANES, _LANES)] = z16
                return c

            lax.fori_loop(0, 1008 // _LANES, zflat_body, 0)
            off = pl.multiple_of(sid * zchunk, 8)

            def initrows(k, c):
                o = pl.multiple_of(off + k * _LANES, 8)
                pltpu.sync_copy(msg_a, sums_sh.at[pl.ds(o, _LANES)])
                return c

            lax.fori_loop(0, zchunk // _LANES, initrows, 0)
            pltpu.sync_copy(msg_a,
                            sums_sh.at[pl.ds(off + zchunk - _LANES, _LANES)])
            pltpu.sync_copy(zflat_v, cnt_sh.at[pl.ds(off, 1008)])

        plsc.subcore_barrier()

        rows = (rows_a, rows_b)
        sems = (sem_a, sem_b)
        msgs = (msg_a, msg_b)
        msems = (sem_ma, sem_mb)

        def issue(gg, buf, s):
            gb = pl.multiple_of(gg * _LANES, _LANES)
            pltpu.async_copy(y_hbm.at[src_v[pl.ds(gb, _LANES)]], buf, s)

        def compute(gg, buf, msg):
            gb = pl.multiple_of(gg * _LANES, _LANES)
            t16 = et_v[pl.ds(gb, _LANES)]
            for c in range(_LANES):
                av = att_v[pl.ds(t16[c] * nb, _LANES)]
                abf = []
                for b in range(nb):
                    af = jnp.full((_LANES,), av[b], jnp.float32)
                    abf.append(plsc.pack(
                        af, af, format=plsc.PackFormat.INTERLEAVED))
                for j in range(d // 32):
                    acc = None
                    for b in range(nb):
                        w = plsc.bitcast(
                            buf[c, pl.ds(b * (d // 2) + j * _LANES,
                                         _LANES)],
                            jnp.bfloat16)
                        acc = abf[b] * w if b == 0 else acc + abf[b] * w
                    pe, po = plsc.unpack(
                        acc, format=plsc.PackFormat.INTERLEAVED)
                    msg[c, pl.ds(j * 32, _LANES)] = pe
                    msg[c, pl.ds(j * 32 + _LANES, _LANES)] = po

        def wait_gather(b):
            pltpu.make_async_copy(
                y_hbm.at[src_v[pl.ds(0, _LANES)]], rows[b], sems[b]).wait()

        def wait_scatter(b):
            pltpu.make_async_copy(
                msgs[b], sums_sh.at[dst_v[pl.ds(0, _LANES)]],
                msems[b]).wait()

        issue(0, rows[0], sems[0])

        def pair(p, carry):
            g = p * 2
            for b in range(2):
                gg = g + b

                @pl.when(gg + 1 < groups)
                def _():
                    issue(gg + 1, rows[1 - b], sems[1 - b])

                wait_gather(b)

                @pl.when(gg >= 2)
                def _():
                    wait_scatter(b)

                compute(gg, rows[b], msgs[b])
                gb = pl.multiple_of(gg * _LANES, _LANES)
                d16 = dst_v[pl.ds(gb, _LANES)]
                pltpu.async_copy(msgs[b], sums_sh.at[d16], msems[b],
                                 add=True)
            return carry

        lax.fori_loop(0, groups // 2, pair, 0)
        wait_scatter(0)
        wait_scatter(1)
        pltpu.sync_copy(ones_v, cnt_sh.at[dst_v], add=True)
        plsc.subcore_barrier()

        @pl.when(sid < 10)
        def _():
            off = pl.multiple_of(sid * zchunk, 8)

            def outrows(k, c):
                o = pl.multiple_of(off + k * _LANES, 8)
                pltpu.sync_copy(sums_sh.at[pl.ds(o, _LANES)], msg_a)
                pltpu.sync_copy(msg_a, sums_out.at[cid, pl.ds(o, _LANES)])
                return c

            lax.fori_loop(0, zchunk // _LANES, outrows, 0)
            o2 = pl.multiple_of(off + zchunk - _LANES, 8)
            pltpu.sync_copy(sums_sh.at[pl.ds(o2, _LANES)], msg_a)
            pltpu.sync_copy(msg_a, sums_out.at[cid, pl.ds(o2, _LANES)])
            coff = pl.multiple_of(cid * n + off, 8)
            pltpu.sync_copy(cnt_sh.at[pl.ds(off, zchunk)],
                            zflat_v.at[pl.ds(0, zchunk)])
            pltpu.sync_copy(zflat_v.at[pl.ds(0, zchunk)],
                            cnt_out.at[pl.ds(coff, zchunk)])

    return sc


def _combine(base, sums, cnt3):
    n, d = base.shape
    rows = 2000
    assert n % rows == 0

    def body(base_ref, s_ref, c_ref, o_ref):
        s = s_ref[0] + s_ref[1]
        c = c_ref[0] + c_ref[1]
        o_ref[...] = base_ref[...] + s / jnp.maximum(c, 1.0)

    return pl.pallas_call(
        body,
        grid=(n // rows,),
        in_specs=[
            pl.BlockSpec((rows, d), lambda i: (i, 0)),
            pl.BlockSpec((2, rows, d), lambda i: (0, i, 0)),
            pl.BlockSpec((2, rows, 1), lambda i: (0, i, 0)),
        ],
        out_specs=pl.BlockSpec((rows, d), lambda i: (i, 0)),
        out_shape=jax.ShapeDtypeStruct((n, d), jnp.float32),
    )(base, sums, cnt3)


def kernel(x, edge_index, edge_type, basis, att, root, bias):
    n, d = x.shape
    nb = basis.shape[0]
    e = edge_type.shape[0]
    nbd = nb * d

    src = edge_index[0].astype(jnp.int32)
    dst = edge_index[1].astype(jnp.int32)
    et = edge_type.astype(jnp.int32)

    w2 = basis.transpose(1, 0, 2).reshape(d, nbd)
    a_idx = (np.arange(0, nbd, 32)[:, None] + np.arange(16)[None, :])
    a_idx = a_idx.reshape(-1)
    w2 = w2[:, np.concatenate([a_idx, a_idx + 16])]
    wcat = jnp.concatenate([w2, root], axis=1)
    bias2 = bias.reshape(1, d)

    yp, base = _project(x, wcat, bias2, nbd)

    e_pad = -(-e // (_N_WORKERS * _LANES * 2)) * (_N_WORKERS * _LANES * 2)
    pad = e_pad - e
    src_p = jnp.concatenate([src, jnp.zeros((pad,), jnp.int32)])
    dst_p = jnp.concatenate([dst, jnp.full((pad,), n, jnp.int32)])
    et_p = jnp.concatenate([et, jnp.zeros((pad,), jnp.int32)])
    att_flat = att.reshape(-1)

    sc_fn = _make_sc_edge_kernel(n, d, nb, att.shape[0], e_pad)
    sums, cnt = sc_fn(yp, src_p, dst_p, et_p, att_flat)

    return _combine(base, sums, cnt.reshape(2, n, 1))
